# baseline (speedup 1.0000x reference)
import functools

import jax
import jax.numpy as jnp
import numpy as np
from jax import lax
from jax.experimental import pallas as pl
from jax.experimental.pallas import tpu as pltpu
from jax.experimental.pallas import tpu_sc as plsc

_N_OPS = (2, 2, 2, 2, 2, 4, 4, 4, 2, 2)
_OFFS = (0, 2, 4, 6, 8, 10, 14, 18, 22, 24)
_RAW_W = 26
_N_PROG = 10
_TILE = 256


def _gelu(x):
    return 0.5 * x * (1.0 + lax.erf(x * np.float32(1.0 / np.sqrt(2.0))))


def _safe(b):
    return jnp.where(b == 0.0, jnp.float32(1.0), b)


def _prog_results(j, ops, floor):
    a, b = ops[0], ops[1]
    zero = jnp.zeros_like(a)
    if j == 0:
        return a + b, zero, zero
    if j == 1:
        return a - b, zero, zero
    if j == 2:
        return a * b, zero, zero
    if j == 3:
        sb = _safe(b)
        return a - floor(a / sb) * sb, zero, zero
    if j == 4:
        return floor(a / _safe(b)), zero, zero
    if j in (5, 6):
        c, d = ops[2], ops[3]
        sgn = 1.0 if j == 5 else -1.0
        tot = a * 3600.0 + b * 60.0 + c + sgn * d
        tot = tot - 86400.0 * floor(tot / 86400.0)
        hh = floor(tot / 3600.0)
        rem = tot - hh * 3600.0
        mm = floor(rem / 60.0)
        ss = rem - mm * 60.0
        return hh, mm, ss
    if j == 7:
        c, d = ops[2], ops[3]
        dm = (c * 60.0 + d) - (a * 60.0 + b)
        dm = dm - 1440.0 * floor(dm / 1440.0)
        mm = floor(dm / 60.0)
        return mm, dm - 60.0 * mm, dm
    if j == 8:
        return (a > b).astype(jnp.float32), zero, zero
    return (a == b).astype(jnp.float32), zero, zero


def _tc_body(x_ref, wr_ref, br_ref, *refs):
    w1_refs = refs[0:10]
    b1_refs = refs[10:20]
    w2_refs = refs[20:30]
    b2_refs = refs[30:40]
    logits_ref, raw_ref = refs[40], refs[41]

    x = x_ref[...]
    logits_ref[...] = (
        jnp.dot(x, wr_ref[...], preferred_element_type=jnp.float32)
        + br_ref[...]
    )
    for j in range(_N_PROG):
        n_ops = _N_OPS[j]
        off = _OFFS[j]
        h = jnp.dot(x, w1_refs[j][...], preferred_element_type=jnp.float32)
        h = _gelu(h + b1_refs[j][...])
        raw_ref[:, off:off + n_ops] = (
            jnp.dot(h, w2_refs[j][...], preferred_element_type=jnp.float32)
            + b2_refs[j][...]
        )


def _stage1(hidden, params):
    n_tok, d_model = hidden.shape
    n_tiles = n_tok // _TILE
    heads = params["heads"]
    names = ["IntAdd", "IntSub", "IntMul", "Mod", "FloorDiv", "TimeAdd",
             "TimeSub", "DurationBetween", "Greater", "Equal"]
    w1s = [heads[n]["w1"] for n in names]
    b1s = [heads[n]["b1"].reshape(1, -1) for n in names]
    w2s = [heads[n]["w2"] for n in names]
    b2s = [heads[n]["b2"].reshape(1, -1) for n in names]
    wr = params["router"]["w"]
    br = params["router"]["b"].reshape(1, -1)
    d_hid = w1s[0].shape[1]

    in_specs = (
        [pl.BlockSpec((_TILE, d_model), lambda i: (i, 0)),
         pl.BlockSpec((d_model, _N_PROG), lambda i: (0, 0)),
         pl.BlockSpec((1, _N_PROG), lambda i: (0, 0))]
        + [pl.BlockSpec((d_model, d_hid), lambda i: (0, 0))] * 10
        + [pl.BlockSpec((1, d_hid), lambda i: (0, 0))] * 10
        + [pl.BlockSpec((d_hid, _N_OPS[j]), lambda i: (0, 0))
           for j in range(10)]
        + [pl.BlockSpec((1, _N_OPS[j]), lambda i: (0, 0)) for j in range(10)]
    )
    out_specs = [
        pl.BlockSpec((_TILE, _N_PROG), lambda i: (i, 0)),
        pl.BlockSpec((_TILE, _RAW_W), lambda i: (i, 0)),
    ]
    out_shapes = [
        jax.ShapeDtypeStruct((n_tok, _N_PROG), jnp.float32),
        jax.ShapeDtypeStruct((n_tok, _RAW_W), jnp.float32),
    ]
    return pl.pallas_call(
        _tc_body,
        grid=(n_tiles,),
        in_specs=in_specs,
        out_specs=out_specs,
        out_shape=out_shapes,
        compiler_params=pltpu.CompilerParams(
            dimension_semantics=("parallel",),
        ),
    )(hidden, wr, br, *w1s, *b1s, *w2s, *b2s)


def _sc_floor(x):
    t = x.astype(jnp.int32).astype(jnp.float32)
    return jnp.where(x < t, t - 1.0, t)


def _sc_round(x):
    fl = _sc_floor(x)
    hi = _sc_floor(x + 0.5)
    tie = (x - fl) == 0.5
    even = (fl.astype(jnp.int32) & 1) == 0
    return jnp.where(tie & even, fl, hi)


def _dispatch_sc(raw, program_ids):
    n_tok = raw.shape[0]
    info = plsc.get_sparse_core_info()
    nc, ns = info.num_cores, info.num_subcores
    chunk = n_tok // (nc * ns)
    n_vec = chunk // 16
    offtab = jnp.array(list(_OFFS) + [0] * 6, dtype=jnp.int32)

    mesh = plsc.VectorSubcoreMesh(core_axis_name="c", subcore_axis_name="s")

    @functools.partial(
        pl.kernel,
        mesh=mesh,
        compiler_params=pltpu.CompilerParams(needs_layout_passes=False),
        out_type=[jax.ShapeDtypeStruct((n_tok,), jnp.float32)] * 3,
        scratch_types=[
            pltpu.VMEM((chunk, _RAW_W), jnp.float32),
            pltpu.VMEM((chunk,), jnp.int32),
            pltpu.VMEM((16,), jnp.int32),
            pltpu.VMEM((chunk,), jnp.float32),
            pltpu.VMEM((chunk,), jnp.float32),
            pltpu.VMEM((chunk,), jnp.float32),
        ],
    )
    def sc_kernel(raw_hbm, pid_hbm, off_hbm, out0, out1, out2,
                  raw_v, pid_v, off_v, r0_v, r1_v, r2_v):
        wid = lax.axis_index("s") * nc + lax.axis_index("c")
        base = wid * chunk
        pltpu.sync_copy(raw_hbm.at[pl.ds(base, chunk)], raw_v)
        pltpu.sync_copy(pid_hbm.at[pl.ds(base, chunk)], pid_v)
        pltpu.sync_copy(off_hbm, off_v)
        for i in range(n_vec):
            pid = pid_v[pl.ds(i * 16, 16)]
            off = 2 * pid + 2 * jnp.clip(pid - 5, 0, 3)
            rows = lax.iota(jnp.int32, 16) + i * 16
            ga = plsc.load_gather(raw_v, [rows, off])
            gb = plsc.load_gather(raw_v, [rows, off + 1])
            gc = plsc.load_gather(raw_v, [rows, jnp.minimum(off + 2, _RAW_W - 1)])
            gd = plsc.load_gather(raw_v, [rows, jnp.minimum(off + 3, _RAW_W - 1)])
            ops = [_sc_round(ga), _sc_round(gb), _sc_round(gc), _sc_round(gd)]
            r0 = jnp.zeros((16,), jnp.float32)
            r1 = jnp.zeros((16,), jnp.float32)
            r2 = jnp.zeros((16,), jnp.float32)
            for j in range(_N_PROG):
                p0, p1, p2 = _prog_results(j, ops, _sc_floor)
                m = pid == j
                r0 = jnp.where(m, p0, r0)
                r1 = jnp.where(m, p1, r1)
                r2 = jnp.where(m, p2, r2)
            r0_v[pl.ds(i * 16, 16)] = r0
            r1_v[pl.ds(i * 16, 16)] = r1
            r2_v[pl.ds(i * 16, 16)] = r2
        pltpu.sync_copy(r0_v, out0.at[pl.ds(base, chunk)])
        pltpu.sync_copy(r1_v, out1.at[pl.ds(base, chunk)])
        pltpu.sync_copy(r2_v, out2.at[pl.ds(base, chunk)])

    r0, r1, r2 = sc_kernel(raw, program_ids.astype(jnp.int32), offtab)
    return jnp.stack([r0, r1, r2], axis=1)


def kernel(hidden, params, program_ids):
    logits, raw = _stage1(hidden, params)
    result = _dispatch_sc(raw, program_ids)
    return (result, logits, raw, program_ids)


# speedup vs baseline: 1.1068x; 1.1068x over previous
import functools

import jax
import jax.numpy as jnp
import numpy as np
from jax import lax
from jax.experimental import pallas as pl
from jax.experimental.pallas import tpu as pltpu
from jax.experimental.pallas import tpu_sc as plsc

_N_OPS = (2, 2, 2, 2, 2, 4, 4, 4, 2, 2)
_OFFS = (0, 2, 4, 6, 8, 10, 14, 18, 22, 24)
_RAW_W = 26
_N_PROG = 10
_TILE = 512


def _gelu(x):
    return 0.5 * x * (1.0 + lax.erf(x * np.float32(1.0 / np.sqrt(2.0))))


def _safe(b):
    return jnp.where(b == 0.0, jnp.float32(1.0), b)


def _prog_results(j, ops, floor):
    a, b = ops[0], ops[1]
    zero = jnp.zeros_like(a)
    if j == 0:
        return a + b, zero, zero
    if j == 1:
        return a - b, zero, zero
    if j == 2:
        return a * b, zero, zero
    if j == 3:
        sb = _safe(b)
        return a - floor(a / sb) * sb, zero, zero
    if j == 4:
        return floor(a / _safe(b)), zero, zero
    if j in (5, 6):
        c, d = ops[2], ops[3]
        sgn = 1.0 if j == 5 else -1.0
        tot = a * 3600.0 + b * 60.0 + c + sgn * d
        tot = tot - 86400.0 * floor(tot / 86400.0)
        hh = floor(tot / 3600.0)
        rem = tot - hh * 3600.0
        mm = floor(rem / 60.0)
        ss = rem - mm * 60.0
        return hh, mm, ss
    if j == 7:
        c, d = ops[2], ops[3]
        dm = (c * 60.0 + d) - (a * 60.0 + b)
        dm = dm - 1440.0 * floor(dm / 1440.0)
        mm = floor(dm / 60.0)
        return mm, dm - 60.0 * mm, dm
    if j == 8:
        return (a > b).astype(jnp.float32), zero, zero
    return (a == b).astype(jnp.float32), zero, zero


def _tc_body(x_ref, wr_ref, br_ref, *refs):
    w1_refs = refs[0:10]
    b1_refs = refs[10:20]
    w2_refs = refs[20:30]
    b2_refs = refs[30:40]
    logits_ref, raw_ref = refs[40], refs[41]

    x = x_ref[...]
    logits_ref[...] = (
        jnp.dot(x, wr_ref[...], preferred_element_type=jnp.float32)
        + br_ref[...]
    )
    for j in range(_N_PROG):
        n_ops = _N_OPS[j]
        off = _OFFS[j]
        h = jnp.dot(x, w1_refs[j][...], preferred_element_type=jnp.float32)
        h = _gelu(h + b1_refs[j][...])
        raw_ref[:, off:off + n_ops] = (
            jnp.dot(h, w2_refs[j][...], preferred_element_type=jnp.float32)
            + b2_refs[j][...]
        )


def _stage1(hidden, params):
    n_tok, d_model = hidden.shape
    n_tiles = n_tok // _TILE
    heads = params["heads"]
    names = ["IntAdd", "IntSub", "IntMul", "Mod", "FloorDiv", "TimeAdd",
             "TimeSub", "DurationBetween", "Greater", "Equal"]
    w1s = [heads[n]["w1"] for n in names]
    b1s = [heads[n]["b1"].reshape(1, -1) for n in names]
    w2s = [heads[n]["w2"] for n in names]
    b2s = [heads[n]["b2"].reshape(1, -1) for n in names]
    wr = params["router"]["w"]
    br = params["router"]["b"].reshape(1, -1)
    d_hid = w1s[0].shape[1]

    in_specs = (
        [pl.BlockSpec((_TILE, d_model), lambda i: (i, 0)),
         pl.BlockSpec((d_model, _N_PROG), lambda i: (0, 0)),
         pl.BlockSpec((1, _N_PROG), lambda i: (0, 0))]
        + [pl.BlockSpec((d_model, d_hid), lambda i: (0, 0))] * 10
        + [pl.BlockSpec((1, d_hid), lambda i: (0, 0))] * 10
        + [pl.BlockSpec((d_hid, _N_OPS[j]), lambda i: (0, 0))
           for j in range(10)]
        + [pl.BlockSpec((1, _N_OPS[j]), lambda i: (0, 0)) for j in range(10)]
    )
    out_specs = [
        pl.BlockSpec((_TILE, _N_PROG), lambda i: (i, 0)),
        pl.BlockSpec((_TILE, _RAW_W), lambda i: (i, 0)),
    ]
    out_shapes = [
        jax.ShapeDtypeStruct((n_tok, _N_PROG), jnp.float32),
        jax.ShapeDtypeStruct((n_tok, _RAW_W), jnp.float32),
    ]
    return pl.pallas_call(
        _tc_body,
        grid=(n_tiles,),
        in_specs=in_specs,
        out_specs=out_specs,
        out_shape=out_shapes,
        compiler_params=pltpu.CompilerParams(
            dimension_semantics=("parallel",),
        ),
    )(hidden, wr, br, *w1s, *b1s, *w2s, *b2s)


def _sc_floor(x):
    t = x.astype(jnp.int32).astype(jnp.float32)
    return jnp.where(x < t, t - 1.0, t)


def _sc_round(x):
    fl = _sc_floor(x)
    hi = _sc_floor(x + 0.5)
    tie = (x - fl) == 0.5
    even = (fl.astype(jnp.int32) & 1) == 0
    return jnp.where(tie & even, fl, hi)


def _dispatch_sc(raw, program_ids):
    n_tok = raw.shape[0]
    info = plsc.get_sparse_core_info()
    nc, ns = info.num_cores, info.num_subcores
    chunk = n_tok // (nc * ns)
    n_vec = chunk // 16

    mesh = plsc.VectorSubcoreMesh(core_axis_name="c", subcore_axis_name="s")

    @functools.partial(
        pl.kernel,
        mesh=mesh,
        compiler_params=pltpu.CompilerParams(needs_layout_passes=False),
        out_type=[jax.ShapeDtypeStruct((n_tok,), jnp.float32)] * 3,
        scratch_types=[
            pltpu.VMEM((chunk, _RAW_W), jnp.float32),
            pltpu.VMEM((chunk,), jnp.int32),
            pltpu.VMEM((chunk,), jnp.float32),
            pltpu.VMEM((chunk,), jnp.float32),
            pltpu.VMEM((chunk,), jnp.float32),
        ],
    )
    def sc_kernel(raw_hbm, pid_hbm, out0, out1, out2,
                  raw_v, pid_v, r0_v, r1_v, r2_v):
        wid = lax.axis_index("s") * nc + lax.axis_index("c")
        base = wid * chunk
        pltpu.sync_copy(raw_hbm.at[pl.ds(base, chunk)], raw_v)
        pltpu.sync_copy(pid_hbm.at[pl.ds(base, chunk)], pid_v)
        for i in range(n_vec):
            pid = pid_v[pl.ds(i * 16, 16)]
            off = 2 * pid + 2 * jnp.clip(pid - 5, 0, 3)
            rows = lax.iota(jnp.int32, 16) + i * 16
            ga = plsc.load_gather(raw_v, [rows, off])
            gb = plsc.load_gather(raw_v, [rows, off + 1])
            gc = plsc.load_gather(raw_v, [rows, jnp.minimum(off + 2, _RAW_W - 1)])
            gd = plsc.load_gather(raw_v, [rows, jnp.minimum(off + 3, _RAW_W - 1)])
            ops = [_sc_round(ga), _sc_round(gb), _sc_round(gc), _sc_round(gd)]
            r0 = jnp.zeros((16,), jnp.float32)
            r1 = jnp.zeros((16,), jnp.float32)
            r2 = jnp.zeros((16,), jnp.float32)
            for j in range(_N_PROG):
                p0, p1, p2 = _prog_results(j, ops, _sc_floor)
                m = pid == j
                r0 = jnp.where(m, p0, r0)
                r1 = jnp.where(m, p1, r1)
                r2 = jnp.where(m, p2, r2)
            r0_v[pl.ds(i * 16, 16)] = r0
            r1_v[pl.ds(i * 16, 16)] = r1
            r2_v[pl.ds(i * 16, 16)] = r2
        pltpu.sync_copy(r0_v, out0.at[pl.ds(base, chunk)])
        pltpu.sync_copy(r1_v, out1.at[pl.ds(base, chunk)])
        pltpu.sync_copy(r2_v, out2.at[pl.ds(base, chunk)])

    r0, r1, r2 = sc_kernel(raw, program_ids.astype(jnp.int32))
    return jnp.stack([r0, r1, r2], axis=1)


def kernel(hidden, params, program_ids):
    logits, raw = _stage1(hidden, params)
    result = _dispatch_sc(raw, program_ids)
    return (result, logits, raw, program_ids)


# speedup vs baseline: 1.1203x; 1.0122x over previous
import functools

import jax
import jax.numpy as jnp
import numpy as np
from jax import lax
from jax.experimental import pallas as pl
from jax.experimental.pallas import tpu as pltpu
from jax.experimental.pallas import tpu_sc as plsc

_N_OPS = (2, 2, 2, 2, 2, 4, 4, 4, 2, 2)
_OFFS = (0, 2, 4, 6, 8, 10, 14, 18, 22, 24)
_RAW_W = 26
_N_PROG = 10
_TILE = 1024


def _gelu(x):
    return 0.5 * x * (1.0 + lax.erf(x * np.float32(1.0 / np.sqrt(2.0))))


def _safe(b):
    return jnp.where(b == 0.0, jnp.float32(1.0), b)


def _prog_results(j, ops, floor):
    a, b = ops[0], ops[1]
    zero = jnp.zeros_like(a)
    if j == 0:
        return a + b, zero, zero
    if j == 1:
        return a - b, zero, zero
    if j == 2:
        return a * b, zero, zero
    if j == 3:
        sb = _safe(b)
        return a - floor(a / sb) * sb, zero, zero
    if j == 4:
        return floor(a / _safe(b)), zero, zero
    if j in (5, 6):
        c, d = ops[2], ops[3]
        sgn = 1.0 if j == 5 else -1.0
        tot = a * 3600.0 + b * 60.0 + c + sgn * d
        tot = tot - 86400.0 * floor(tot / 86400.0)
        hh = floor(tot / 3600.0)
        rem = tot - hh * 3600.0
        mm = floor(rem / 60.0)
        ss = rem - mm * 60.0
        return hh, mm, ss
    if j == 7:
        c, d = ops[2], ops[3]
        dm = (c * 60.0 + d) - (a * 60.0 + b)
        dm = dm - 1440.0 * floor(dm / 1440.0)
        mm = floor(dm / 60.0)
        return mm, dm - 60.0 * mm, dm
    if j == 8:
        return (a > b).astype(jnp.float32), zero, zero
    return (a == b).astype(jnp.float32), zero, zero


def _tc_body(x_ref, wr_ref, br_ref, *refs):
    w1_refs = refs[0:10]
    b1_refs = refs[10:20]
    w2_refs = refs[20:30]
    b2_refs = refs[30:40]
    logits_ref, raw_ref = refs[40], refs[41]

    x = x_ref[...]
    logits_ref[...] = (
        jnp.dot(x, wr_ref[...], preferred_element_type=jnp.float32)
        + br_ref[...]
    )
    for j in range(_N_PROG):
        n_ops = _N_OPS[j]
        off = _OFFS[j]
        h = jnp.dot(x, w1_refs[j][...], preferred_element_type=jnp.float32)
        h = _gelu(h + b1_refs[j][...])
        raw_ref[:, off:off + n_ops] = (
            jnp.dot(h, w2_refs[j][...], preferred_element_type=jnp.float32)
            + b2_refs[j][...]
        )


def _stage1(hidden, params):
    n_tok, d_model = hidden.shape
    n_tiles = n_tok // _TILE
    heads = params["heads"]
    names = ["IntAdd", "IntSub", "IntMul", "Mod", "FloorDiv", "TimeAdd",
             "TimeSub", "DurationBetween", "Greater", "Equal"]
    w1s = [heads[n]["w1"] for n in names]
    b1s = [heads[n]["b1"].reshape(1, -1) for n in names]
    w2s = [heads[n]["w2"] for n in names]
    b2s = [heads[n]["b2"].reshape(1, -1) for n in names]
    wr = params["router"]["w"]
    br = params["router"]["b"].reshape(1, -1)
    d_hid = w1s[0].shape[1]

    in_specs = (
        [pl.BlockSpec((_TILE, d_model), lambda i: (i, 0)),
         pl.BlockSpec((d_model, _N_PROG), lambda i: (0, 0)),
         pl.BlockSpec((1, _N_PROG), lambda i: (0, 0))]
        + [pl.BlockSpec((d_model, d_hid), lambda i: (0, 0))] * 10
        + [pl.BlockSpec((1, d_hid), lambda i: (0, 0))] * 10
        + [pl.BlockSpec((d_hid, _N_OPS[j]), lambda i: (0, 0))
           for j in range(10)]
        + [pl.BlockSpec((1, _N_OPS[j]), lambda i: (0, 0)) for j in range(10)]
    )
    out_specs = [
        pl.BlockSpec((_TILE, _N_PROG), lambda i: (i, 0)),
        pl.BlockSpec((_TILE, _RAW_W), lambda i: (i, 0)),
    ]
    out_shapes = [
        jax.ShapeDtypeStruct((n_tok, _N_PROG), jnp.float32),
        jax.ShapeDtypeStruct((n_tok, _RAW_W), jnp.float32),
    ]
    return pl.pallas_call(
        _tc_body,
        grid=(n_tiles,),
        in_specs=in_specs,
        out_specs=out_specs,
        out_shape=out_shapes,
        compiler_params=pltpu.CompilerParams(
            dimension_semantics=("parallel",),
            vmem_limit_bytes=120 * 1024 * 1024,
        ),
    )(hidden, wr, br, *w1s, *b1s, *w2s, *b2s)


def _sc_floor(x):
    t = x.astype(jnp.int32).astype(jnp.float32)
    return jnp.where(x < t, t - 1.0, t)


def _sc_round(x):
    fl = _sc_floor(x)
    hi = _sc_floor(x + 0.5)
    tie = (x - fl) == 0.5
    even = (fl.astype(jnp.int32) & 1) == 0
    return jnp.where(tie & even, fl, hi)


def _dispatch_sc(raw, program_ids):
    n_tok = raw.shape[0]
    info = plsc.get_sparse_core_info()
    nc, ns = info.num_cores, info.num_subcores
    chunk = n_tok // (nc * ns)
    n_vec = chunk // 16

    mesh = plsc.VectorSubcoreMesh(core_axis_name="c", subcore_axis_name="s")

    @functools.partial(
        pl.kernel,
        mesh=mesh,
        compiler_params=pltpu.CompilerParams(needs_layout_passes=False),
        out_type=[jax.ShapeDtypeStruct((n_tok,), jnp.float32)] * 3,
        scratch_types=[
            pltpu.VMEM((chunk, _RAW_W), jnp.float32),
            pltpu.VMEM((chunk,), jnp.int32),
            pltpu.VMEM((chunk,), jnp.float32),
            pltpu.VMEM((chunk,), jnp.float32),
            pltpu.VMEM((chunk,), jnp.float32),
        ],
    )
    def sc_kernel(raw_hbm, pid_hbm, out0, out1, out2,
                  raw_v, pid_v, r0_v, r1_v, r2_v):
        wid = lax.axis_index("s") * nc + lax.axis_index("c")
        base = wid * chunk
        pltpu.sync_copy(raw_hbm.at[pl.ds(base, chunk)], raw_v)
        pltpu.sync_copy(pid_hbm.at[pl.ds(base, chunk)], pid_v)
        for i in range(n_vec):
            pid = pid_v[pl.ds(i * 16, 16)]
            off = 2 * pid + 2 * jnp.clip(pid - 5, 0, 3)
            rows = lax.iota(jnp.int32, 16) + i * 16
            ga = plsc.load_gather(raw_v, [rows, off])
            gb = plsc.load_gather(raw_v, [rows, off + 1])
            gc = plsc.load_gather(raw_v, [rows, jnp.minimum(off + 2, _RAW_W - 1)])
            gd = plsc.load_gather(raw_v, [rows, jnp.minimum(off + 3, _RAW_W - 1)])
            ops = [_sc_round(ga), _sc_round(gb), _sc_round(gc), _sc_round(gd)]
            r0 = jnp.zeros((16,), jnp.float32)
            r1 = jnp.zeros((16,), jnp.float32)
            r2 = jnp.zeros((16,), jnp.float32)
            for j in range(_N_PROG):
                p0, p1, p2 = _prog_results(j, ops, _sc_floor)
                m = pid == j
                r0 = jnp.where(m, p0, r0)
                r1 = jnp.where(m, p1, r1)
                r2 = jnp.where(m, p2, r2)
            r0_v[pl.ds(i * 16, 16)] = r0
            r1_v[pl.ds(i * 16, 16)] = r1
            r2_v[pl.ds(i * 16, 16)] = r2
        pltpu.sync_copy(r0_v, out0.at[pl.ds(base, chunk)])
        pltpu.sync_copy(r1_v, out1.at[pl.ds(base, chunk)])
        pltpu.sync_copy(r2_v, out2.at[pl.ds(base, chunk)])

    r0, r1, r2 = sc_kernel(raw, program_ids.astype(jnp.int32))
    return jnp.stack([r0, r1, r2], axis=1)


def kernel(hidden, params, program_ids):
    logits, raw = _stage1(hidden, params)
    result = _dispatch_sc(raw, program_ids)
    return (result, logits, raw, program_ids)
